# baseline (speedup 1.0000x reference)
import functools

import jax
import jax.numpy as jnp
from jax import lax
from jax.experimental import pallas as pl
from jax.experimental.pallas import tpu as pltpu
from jax.experimental.pallas import tpu_sc as plsc

_NC = 2
_NS = 16
_NW = _NC * _NS
_CHUNK = 32768


def _sc_copy(n_total: int):
    per_w = n_total // _NW
    n_chunks = per_w // _CHUNK
    mesh = plsc.VectorSubcoreMesh(core_axis_name="c", subcore_axis_name="s")

    @functools.partial(
        pl.kernel,
        mesh=mesh,
        out_type=jax.ShapeDtypeStruct((n_total,), jnp.float32),
        scratch_types=[
            pltpu.VMEM((_CHUNK,), jnp.float32),
        ],
    )
    def k(w_hbm, out_hbm, buf):
        wid = lax.axis_index("s") * _NC + lax.axis_index("c")
        base = wid * per_w
        for j in range(n_chunks):
            off = base + j * _CHUNK
            pltpu.sync_copy(w_hbm.at[pl.ds(off, _CHUNK)], buf)
            pltpu.sync_copy(buf, out_hbm.at[pl.ds(off, _CHUNK)])

    return k


def kernel(x, weight):
    seq_len = x.shape[1]
    d_model = weight.shape[2]
    n_total = seq_len * d_model
    flat = weight.reshape(-1)
    out = _sc_copy(n_total)(flat)
    return out.reshape(1, seq_len, d_model)


# speedup vs baseline: 1.0291x; 1.0291x over previous
import functools

import jax
import jax.numpy as jnp
from jax import lax
from jax.experimental import pallas as pl
from jax.experimental.pallas import tpu as pltpu
from jax.experimental.pallas import tpu_sc as plsc

_NC = 2
_NS = 16
_NW = _NC * _NS
_CHUNK = 32768


def _sc_copy(n_total: int):
    per_w = n_total // _NW
    n_chunks = per_w // _CHUNK
    mesh = plsc.VectorSubcoreMesh(core_axis_name="c", subcore_axis_name="s")

    @functools.partial(
        pl.kernel,
        mesh=mesh,
        out_type=jax.ShapeDtypeStruct((n_total,), jnp.float32),
        scratch_types=[
            pltpu.VMEM((2, _CHUNK), jnp.float32),
            pltpu.SemaphoreType.DMA((n_chunks,)),
            pltpu.SemaphoreType.DMA((n_chunks,)),
        ],
    )
    def k(w_hbm, out_hbm, buf, in_sems, out_sems):
        wid = lax.axis_index("s") * _NC + lax.axis_index("c")
        base = wid * per_w

        def start_in(j):
            off = base + j * _CHUNK
            return pltpu.async_copy(
                w_hbm.at[pl.ds(off, _CHUNK)], buf.at[j % 2], in_sems.at[j]
            )

        def start_out(j):
            off = base + j * _CHUNK
            return pltpu.async_copy(
                buf.at[j % 2], out_hbm.at[pl.ds(off, _CHUNK)], out_sems.at[j]
            )

        ins = [None] * n_chunks
        outs = [None] * n_chunks
        ins[0] = start_in(0)
        for j in range(n_chunks):
            if j + 1 < n_chunks:
                if j >= 1:
                    outs[j - 1].wait()
                ins[j + 1] = start_in(j + 1)
            ins[j].wait()
            outs[j] = start_out(j)
        for j in range(max(0, n_chunks - 2), n_chunks):
            outs[j].wait()

    return k


def kernel(x, weight):
    seq_len = x.shape[1]
    d_model = weight.shape[2]
    n_total = seq_len * d_model
    flat = weight.reshape(-1)
    out = _sc_copy(n_total)(flat)
    return out.reshape(1, seq_len, d_model)
